# Initial kernel scaffold; baseline (speedup 1.0000x reference)
import jax
import jax.numpy as jnp
from jax.experimental import pallas as pl


def kernel(input_ids, table, W_a, W_c, b_c):
    raise NotImplementedError("write your pallas kernel here")


# speedup vs baseline: 2.9779x; 2.9779x over previous
import functools

import jax
import jax.numpy as jnp
from jax import lax
from jax.experimental import pallas as pl
from jax.experimental.pallas import tpu as pltpu
from jax.experimental.pallas import tpu_sc as plsc

VOCAB = 1000000
HIDDEN = 64
BATCH = 16384
SEQ = 50

NC = 2
NS = 16
NW = NC * NS
BPW = BATCH // NW
CHUNK = BPW * SEQ

TC_BLK = 16384


def _tc_project(w_a_ref, w_c_ref, b_c_ref, t_ref, p0_ref, p1_ref):
    w2 = jnp.dot(w_a_ref[...], w_c_ref[...], preferred_element_type=jnp.float32)
    w2t = w2.T * (1.0 / SEQ)
    t = t_ref[...]
    res = lax.dot_general(w2t, t, (((1,), (1,)), ((), ())),
                          preferred_element_type=jnp.float32)
    b2 = b_c_ref[...] * (1.0 / SEQ)
    p0_ref[...] = res[0:1, :] + b2[0]
    p1_ref[...] = res[1:2, :] + b2[1]


def _project_table(table, w_a, w_c, b_c):
    grid = pl.cdiv(VOCAB, TC_BLK)
    p0, p1 = pl.pallas_call(
        _tc_project,
        grid=(grid,),
        in_specs=[
            pl.BlockSpec((HIDDEN, 3), lambda i: (0, 0)),
            pl.BlockSpec((3, 2), lambda i: (0, 0)),
            pl.BlockSpec((2,), lambda i: (0,)),
            pl.BlockSpec((TC_BLK, HIDDEN), lambda i: (i, 0)),
        ],
        out_specs=[
            pl.BlockSpec((1, TC_BLK), lambda i: (0, i)),
            pl.BlockSpec((1, TC_BLK), lambda i: (0, i)),
        ],
        out_shape=[
            jax.ShapeDtypeStruct((1, VOCAB), jnp.float32),
            jax.ShapeDtypeStruct((1, VOCAB), jnp.float32),
        ],
        compiler_params=pltpu.CompilerParams(
            dimension_semantics=("arbitrary",),
        ),
    )(w_a, w_c, b_c, table)
    return p0.reshape(VOCAB), p1.reshape(VOCAB)


def _accumulate(g_ref, acc_ref):
    for t in range(BPW // 16):
        def body(l, a):
            return a + g_ref[pl.ds(l * BPW + t * 16, 16)]
        acc = lax.fori_loop(0, SEQ, body, jnp.zeros((16,), jnp.float32))
        acc_ref[pl.ds(t * 16, 16)] = acc


def _sc_body(p0_hbm, p1_hbm, ids_hbm, out0_hbm, out1_hbm,
             idx_v, g0_v, g1_v, acc_v, sem0, sem1):
    c = lax.axis_index("c")
    s = lax.axis_index("s")
    w = c * NS + s
    pltpu.sync_copy(ids_hbm.at[w], idx_v)
    cp0 = pltpu.async_copy(p0_hbm.at[idx_v], g0_v, sem0)
    cp1 = pltpu.async_copy(p1_hbm.at[idx_v], g1_v, sem1)
    cp0.wait()
    _accumulate(g0_v, acc_v)
    pltpu.sync_copy(acc_v, out0_hbm.at[pl.ds(w * BPW, BPW)])
    cp1.wait()
    _accumulate(g1_v, acc_v)
    pltpu.sync_copy(acc_v, out1_hbm.at[pl.ds(w * BPW, BPW)])


def _gather_sum(p0, p1, ids_r):
    mesh = plsc.VectorSubcoreMesh(core_axis_name="c", subcore_axis_name="s")
    fn = pl.kernel(
        _sc_body,
        out_type=[
            jax.ShapeDtypeStruct((BATCH,), jnp.float32),
            jax.ShapeDtypeStruct((BATCH,), jnp.float32),
        ],
        mesh=mesh,
        scratch_types=[
            pltpu.VMEM((CHUNK,), jnp.int32),
            pltpu.VMEM((CHUNK,), jnp.float32),
            pltpu.VMEM((CHUNK,), jnp.float32),
            pltpu.VMEM((BPW,), jnp.float32),
            pltpu.SemaphoreType.DMA,
            pltpu.SemaphoreType.DMA,
        ],
    )
    return fn(p0, p1, ids_r)


@jax.jit
def kernel(input_ids, table, W_a, W_c, b_c):
    p0, p1 = _project_table(table, W_a, W_c, b_c)
    ids_r = (input_ids.astype(jnp.int32)
             .reshape(NW, BPW, SEQ)
             .transpose(0, 2, 1)
             .reshape(NW, CHUNK))
    out0, out1 = _gather_sum(p0, p1, ids_r)
    return jnp.stack([out0, out1], axis=1)
